# baseline (speedup 1.0000x reference)
import functools

import jax
import jax.numpy as jnp
from jax import lax
from jax.experimental import pallas as pl
from jax.experimental.pallas import tpu as pltpu
from jax.experimental.pallas import tpu_sc as plsc

BATCH = 4096
SEQ = 50
DIM = 32
NC = 2
NS = 16
NW = NC * NS
BBLK = BATCH // NW
WIDE = 128
NUM_ROWS4 = 250000
NBUF = 3
L = 16

_mesh = plsc.VectorSubcoreMesh(core_axis_name="c", subcore_axis_name="s")


@functools.partial(
    pl.kernel,
    mesh=_mesh,
    out_type=jax.ShapeDtypeStruct((SEQ, DIM, BATCH), jnp.float32),
    scratch_types=[
        pltpu.VMEM((SEQ, BBLK), jnp.int32),
        pltpu.VMEM((SEQ, BBLK), jnp.int32),
        pltpu.VMEM((NBUF, BBLK, WIDE), jnp.float32),
        pltpu.VMEM((2, DIM, BBLK), jnp.float32),
        pltpu.SemaphoreType.DMA,
        pltpu.SemaphoreType.DMA,
    ],
    compiler_params=pltpu.CompilerParams(needs_layout_passes=False),
)
def _gather_kernel(xt_hbm, tab_hbm, out_hbm, idx_v, idx4_v, rows_v, trans_v,
                   gsem, wsem):
    wid = lax.axis_index("s") * NC + lax.axis_index("c")
    b0 = wid * BBLK
    pltpu.sync_copy(xt_hbm.at[:, pl.ds(b0, BBLK)], idx_v)

    def idx_body(i, carry):
        r = lax.rem(i, SEQ)
        c = lax.div(i, SEQ) * L
        v = idx_v[r, pl.ds(c, L)]
        idx4_v[r, pl.ds(c, L)] = lax.shift_right_logical(v, 2)
        return carry
    lax.fori_loop(0, SEQ * (BBLK // L), idx_body, 0)

    for k in range(NBUF - 1):
        pltpu.async_copy(tab_hbm.at[idx4_v.at[k]], rows_v.at[k], gsem)

    iota = lax.iota(jnp.int32, L)

    def body(s, carry):
        @pl.when(s < SEQ)
        def _():
            pltpu.async_copy(
                tab_hbm.at[idx4_v.at[s]], rows_v.at[lax.rem(s, NBUF)], gsem
            )
        o = s - (NBUF - 1)
        pltpu.make_async_copy(
            tab_hbm.at[idx4_v.at[0]], rows_v.at[0], gsem
        ).wait()
        @pl.when(o >= 2)
        def _():
            pltpu.make_async_copy(
                trans_v.at[0], out_hbm.at[0, :, pl.ds(b0, BBLK)], wsem
            ).wait()
        ob = lax.rem(o, NBUF)
        obuf = jnp.broadcast_to(ob, (L,))
        tb = lax.rem(o, 2)
        for k in range(BBLK // L):
            b_vec = iota + (k * L)
            q = jnp.bitwise_and(idx_v[o, pl.ds(k * L, L)], 3)
            qc = lax.shift_left(q, 5)
            for c in range(DIM):
                v = plsc.load_gather(rows_v, [obuf, b_vec, qc + c])
                trans_v[tb, c, pl.ds(k * L, L)] = v
        pltpu.async_copy(
            trans_v.at[tb], out_hbm.at[o, :, pl.ds(b0, BBLK)], wsem
        )
        return carry

    lax.fori_loop(NBUF - 1, SEQ + NBUF - 1, body, 0)

    for _ in range(2):
        pltpu.make_async_copy(
            trans_v.at[0], out_hbm.at[0, :, pl.ds(b0, BBLK)], wsem
        ).wait()


def kernel(x, weight):
    out = _gather_kernel(x.T, weight.reshape(NUM_ROWS4, WIDE))
    return out.transpose(2, 0, 1)
